# baseline (speedup 1.0000x reference)
import functools

import jax
import jax.numpy as jnp
from jax import lax
from jax.experimental import pallas as pl
from jax.experimental.pallas import tpu as pltpu
from jax.experimental.pallas import tpu_sc as plsc

_B, _D = 16384, 512
_M, _K, _d = 8, 1024, 64
_DEAD_THR = 0.0001
_BETA = 0.25

_NSEG = 2
_BSEG = _B // _NSEG

_BLK = 512
_GRID = _BSEG // _BLK

_NB = _B * _M
_NBSEG = _NB // _NSEG
_NW = 32
_ROWS_W = _NBSEG // _NW
_CH = 128


def _tc_body(last, seg, z_ref, cbd_ref, cnt_in_ref, csum_in_ref,
             idx_ref, fidx_ref, cnt_out_ref, csum_out_ref,
             commit_ref, perp_ref, dead_ref, usage_scr, e2_scr, csum_scr):
    pid = pl.program_id(0)

    @pl.when(pid == 0)
    def _init():
        usage_scr[...] = jnp.zeros_like(usage_scr)
        csum_scr[0, 0] = 0.0
        for m in range(_M):
            cb2m = cbd_ref[m // 4][:, (m % 4) * _K:(m % 4 + 1) * _K]
            e2_scr[m:m + 1, :] = 0.25 * jnp.sum(cb2m * cb2m, axis=0,
                                                keepdims=True)

    iota_k = lax.broadcasted_iota(jnp.int32, (_K, 2), 0)
    hi_lo = jnp.concatenate(
        [(iota_k[:, :1] // 8) * 8, iota_k[:, 1:] % 8],
        axis=1).astype(jnp.bfloat16)
    ones_row = jnp.ones((1, _BLK), jnp.bfloat16)
    idx_cols = []
    maxv_acc = jnp.zeros((_BLK, 1), jnp.float32)
    commit_part = 0.0
    xe2_packs = []
    for p in range(_M // 4):
        zp = z_ref[:, p * 4 * _d:(p + 1) * 4 * _d]
        xe2_packs.append(
            lax.dot(zp, cbd_ref[p], precision=lax.Precision.DEFAULT,
                    preferred_element_type=jnp.float32))
    for m in range(_M):
        xe2 = xe2_packs[m // 4][:, (m % 4) * _K:(m % 4 + 1) * _K]
        sim = xe2 - e2_scr[m:m + 1, :]
        maxv = jnp.max(sim, axis=1, keepdims=True)
        onehot = (sim >= maxv).astype(jnp.bfloat16)
        hl = lax.dot(onehot, hi_lo, precision=lax.Precision.DEFAULT,
                     preferred_element_type=jnp.float32)
        idxf = hl[:, :1] + hl[:, 1:]
        idx_cols.append(idxf.astype(jnp.int32))
        cnts = lax.dot(ones_row, onehot, precision=lax.Precision.DEFAULT,
                       preferred_element_type=jnp.float32)
        usage_scr[m:m + 1, :] = usage_scr[m:m + 1, :] + cnts
        maxv_acc = maxv_acc + maxv

    zb = z_ref[...]
    commit_part = jnp.sum(zb * zb) - jnp.sum(maxv_acc)
    idx_blk = jnp.concatenate(idx_cols, axis=1)
    idx_ref[...] = idx_blk
    fidx_ref[...] = idx_blk + lax.broadcasted_iota(
        jnp.int32, (_BLK, _M), 1) * _K
    csum_scr[0, 0] = csum_scr[0, 0] + commit_part

    @pl.when(pid == _GRID - 1)
    def _fin():
        counts = usage_scr[...]
        csum = csum_scr[0, 0]
        if seg > 0:
            counts = counts + cnt_in_ref[...]
            csum = csum + csum_in_ref[0, 0]
        cnt_out_ref[...] = counts
        csum_out_ref[...] = jnp.full((1, 1), csum, jnp.float32)
        if last:
            usage = counts * (1.0 / _B)
            ent = -jnp.sum(usage * jnp.log(jnp.maximum(usage, 1e-12)),
                           axis=1, keepdims=True)
            perp_ref[...] = jnp.exp(ent)
            dead_ref[...] = jnp.mean(
                (usage < _DEAD_THR).astype(jnp.float32),
                axis=1, keepdims=True)
            commit_ref[...] = jnp.full(
                (1, 1), csum * (_BETA / (_M * _B * _d)), jnp.float32)


def _tc_call(z, cbd, cnt_in, csum_in, seg, last):
    stats_shapes = [
        jax.ShapeDtypeStruct((1, 1), jnp.float32),
        jax.ShapeDtypeStruct((_M, 1), jnp.float32),
        jax.ShapeDtypeStruct((_M, 1), jnp.float32),
    ] if last else []
    stats_specs = [
        pl.BlockSpec((1, 1), lambda i: (0, 0)),
        pl.BlockSpec((_M, 1), lambda i: (0, 0)),
        pl.BlockSpec((_M, 1), lambda i: (0, 0)),
    ] if last else []

    def body(z_ref, cbd_ref, cnt_in_ref, csum_in_ref, idx_ref, fidx_ref,
             cnt_out_ref, csum_out_ref, *rest):
        if last:
            commit_ref, perp_ref, dead_ref = rest[:3]
            scr = rest[3:]
        else:
            commit_ref = perp_ref = dead_ref = None
            scr = rest
        _tc_body(last, seg, z_ref, cbd_ref, cnt_in_ref, csum_in_ref,
                 idx_ref, fidx_ref, cnt_out_ref, csum_out_ref,
                 commit_ref, perp_ref, dead_ref, *scr)

    return pl.pallas_call(
        body,
        grid=(_GRID,),
        in_specs=[
            pl.BlockSpec((_BLK, _D), lambda i, s=seg: (i + s * _GRID, 0)),
            pl.BlockSpec((2, 4 * _d, 4 * _K), lambda i: (0, 0, 0)),
            pl.BlockSpec((_M, _K), lambda i: (0, 0)),
            pl.BlockSpec((1, 1), lambda i: (0, 0)),
        ],
        out_specs=[
            pl.BlockSpec((_BLK, _M), lambda i: (i, 0)),
            pl.BlockSpec((_BLK, _M), lambda i: (i, 0)),
            pl.BlockSpec((_M, _K), lambda i: (0, 0)),
            pl.BlockSpec((1, 1), lambda i: (0, 0)),
        ] + stats_specs,
        out_shape=[
            jax.ShapeDtypeStruct((_BSEG, _M), jnp.int32),
            jax.ShapeDtypeStruct((_BSEG, _M), jnp.int32),
            jax.ShapeDtypeStruct((_M, _K), jnp.float32),
            jax.ShapeDtypeStruct((1, 1), jnp.float32),
        ] + stats_shapes,
        scratch_shapes=[
            pltpu.VMEM((_M, _K), jnp.float32),
            pltpu.VMEM((_M, _K), jnp.float32),
            pltpu.SMEM((1, 1), jnp.float32),
        ],
    )(z, cbd, cnt_in, csum_in)


@functools.cache
def _make_sc_gather(seg):
    mesh = plsc.VectorSubcoreMesh(core_axis_name="c", subcore_axis_name="s")
    nchunks = _ROWS_W // _CH
    nbuf = 4

    @functools.partial(
        pl.kernel, mesh=mesh,
        out_type=(),
        compiler_params=pltpu.CompilerParams(use_tc_tiling_on_sc=False),
        scratch_types=[
            pltpu.VMEM((_ROWS_W,), jnp.int32),
            [pltpu.VMEM((_CH, _d), jnp.float32) for _ in range(nbuf)],
            [pltpu.SemaphoreType.DMA for _ in range(nbuf)],
            [pltpu.SemaphoreType.DMA for _ in range(nbuf)],
        ],
    )
    def _sc_gather(table_hbm, idx_hbm, out_hbm, idx_v, rows, gsem, osem):
        wid = lax.axis_index("s") * 2 + lax.axis_index("c")
        ibase = wid * _ROWS_W
        obase = seg * _NBSEG + wid * _ROWS_W
        pltpu.sync_copy(idx_hbm.at[pl.ds(ibase, _ROWS_W)], idx_v)

        def gather(c, j):
            return pltpu.async_copy(
                table_hbm.at[idx_v.at[pl.ds(c * _CH, _CH)]], rows[j],
                gsem[j])

        def put(c, j):
            off = pl.multiple_of(obase + c * _CH, _CH)
            return pltpu.async_copy(rows[j], out_hbm.at[pl.ds(off, _CH)],
                                    osem[j])

        g = [gather(j, j) for j in range(nbuf)]
        o = [None] * nbuf
        for c in range(nchunks):
            j = c % nbuf
            g[j].wait()
            o[j] = put(c, j)
            n = c + nbuf
            if n < nchunks:
                o[j].wait()
                g[j] = gather(n, j)
        for j in range(nbuf):
            if o[j] is not None:
                o[j].wait()

    return _sc_gather


def kernel(z, codebook):
    cbt2 = jnp.transpose(codebook, (0, 2, 1)) * 2.0
    cbd = jnp.stack([
        jnp.concatenate([
            jnp.pad(cbt2[4 * p + j],
                    ((64 * j, 192 - 64 * j), (0, 0)))
            for j in range(4)
        ], axis=1)
        for p in range(2)
    ])
    table = codebook.reshape(_M * _K, _d)

    zq_ref = jax.new_ref(jnp.zeros((_NB, _d), jnp.float32))
    cnt = jnp.zeros((_M, _K), jnp.float32)
    cs = jnp.zeros((1, 1), jnp.float32)
    idx_parts = []
    for s in range(_NSEG):
        last = s == _NSEG - 1
        outs = _tc_call(z, cbt2, cnt, cs, seg=s, last=last)
        idx_s, fidx_s, cnt, cs = outs[:4]
        idx_parts.append(idx_s)
        if last:
            commit, perp, dead = outs[4:]
        _make_sc_gather(s)(table, fidx_s.reshape(_NBSEG), zq_ref)

    z_st = zq_ref[...].reshape(_B, _D)
    idx = jnp.concatenate(idx_parts, axis=0)
    return (z_st, idx, commit.reshape(()), perp.reshape(_M),
            dead.reshape(_M))


# speedup vs baseline: 1.6483x; 1.6483x over previous
import functools

import jax
import jax.numpy as jnp
from jax import lax
from jax.experimental import pallas as pl
from jax.experimental.pallas import tpu as pltpu
from jax.experimental.pallas import tpu_sc as plsc

_B, _D = 16384, 512
_M, _K, _d = 8, 1024, 64
_DEAD_THR = 0.0001
_BETA = 0.25

_NSEG = 2
_BSEG = _B // _NSEG

_BLK = 512
_GRID = _BSEG // _BLK

_NB = _B * _M
_NBSEG = _NB // _NSEG
_NW = 32
_ROWS_W = _NBSEG // _NW
_CH = 128


def _tc_body(last, seg, z_ref, cbd_ref, cnt_in_ref, csum_in_ref,
             idx_ref, fidx_ref, cnt_out_ref, csum_out_ref,
             commit_ref, perp_ref, dead_ref, usage_scr, e2_scr, csum_scr):
    pid = pl.program_id(0)

    @pl.when(pid == 0)
    def _init():
        usage_scr[...] = jnp.zeros_like(usage_scr)
        csum_scr[0, 0] = 0.0
        for m in range(_M):
            cb2m = cbd_ref[m]
            e2_scr[m:m + 1, :] = 0.25 * jnp.sum(cb2m * cb2m, axis=0,
                                                keepdims=True)

    iota_k = lax.broadcasted_iota(jnp.int32, (_K, 2), 0)
    hi_lo = jnp.concatenate(
        [(iota_k[:, :1] // 8) * 8, iota_k[:, 1:] % 8],
        axis=1).astype(jnp.bfloat16)
    ones_row = jnp.ones((1, _BLK), jnp.bfloat16)
    idx_cols = []
    maxv_acc = jnp.zeros((_BLK, 1), jnp.float32)
    for m in range(_M):
        zm = z_ref[:, m * _d:(m + 1) * _d]
        xe2 = lax.dot(zm, cbd_ref[m], precision=lax.Precision.DEFAULT,
                      preferred_element_type=jnp.float32)
        sim = xe2 - e2_scr[m:m + 1, :]
        maxv = jnp.max(sim, axis=1, keepdims=True)
        onehot = (sim >= maxv).astype(jnp.bfloat16)
        hl = lax.dot(onehot, hi_lo, precision=lax.Precision.DEFAULT,
                     preferred_element_type=jnp.float32)
        idxf = hl[:, :1] + hl[:, 1:]
        idx_cols.append(idxf.astype(jnp.int32))
        cnts = lax.dot(ones_row, onehot, precision=lax.Precision.DEFAULT,
                       preferred_element_type=jnp.float32)
        usage_scr[m:m + 1, :] = usage_scr[m:m + 1, :] + cnts
        maxv_acc = maxv_acc + maxv

    zb = z_ref[...]
    commit_part = jnp.sum(zb * zb) - jnp.sum(maxv_acc)
    idx_blk = jnp.concatenate(idx_cols, axis=1)
    idx_ref[...] = idx_blk
    fidx_ref[...] = idx_blk + lax.broadcasted_iota(
        jnp.int32, (_BLK, _M), 1) * _K
    csum_scr[0, 0] = csum_scr[0, 0] + commit_part

    @pl.when(pid == _GRID - 1)
    def _fin():
        counts = usage_scr[...]
        csum = csum_scr[0, 0]
        if seg > 0:
            counts = counts + cnt_in_ref[...]
            csum = csum + csum_in_ref[0, 0]
        cnt_out_ref[...] = counts
        csum_out_ref[...] = jnp.full((1, 1), csum, jnp.float32)
        if last:
            usage = counts * (1.0 / _B)
            ent = -jnp.sum(usage * jnp.log(jnp.maximum(usage, 1e-12)),
                           axis=1, keepdims=True)
            perp_ref[...] = jnp.exp(ent)
            dead_ref[...] = jnp.mean(
                (usage < _DEAD_THR).astype(jnp.float32),
                axis=1, keepdims=True)
            commit_ref[...] = jnp.full(
                (1, 1), csum * (_BETA / (_M * _B * _d)), jnp.float32)


def _tc_call(z, cbd, cnt_in, csum_in, seg, last):
    stats_shapes = [
        jax.ShapeDtypeStruct((1, 1), jnp.float32),
        jax.ShapeDtypeStruct((_M, 1), jnp.float32),
        jax.ShapeDtypeStruct((_M, 1), jnp.float32),
    ] if last else []
    stats_specs = [
        pl.BlockSpec((1, 1), lambda i: (0, 0)),
        pl.BlockSpec((_M, 1), lambda i: (0, 0)),
        pl.BlockSpec((_M, 1), lambda i: (0, 0)),
    ] if last else []

    def body(z_ref, cbd_ref, cnt_in_ref, csum_in_ref, idx_ref, fidx_ref,
             cnt_out_ref, csum_out_ref, *rest):
        if last:
            commit_ref, perp_ref, dead_ref = rest[:3]
            scr = rest[3:]
        else:
            commit_ref = perp_ref = dead_ref = None
            scr = rest
        _tc_body(last, seg, z_ref, cbd_ref, cnt_in_ref, csum_in_ref,
                 idx_ref, fidx_ref, cnt_out_ref, csum_out_ref,
                 commit_ref, perp_ref, dead_ref, *scr)

    return pl.pallas_call(
        body,
        grid=(_GRID,),
        in_specs=[
            pl.BlockSpec((_BLK, _D), lambda i, s=seg: (i + s * _GRID, 0)),
            pl.BlockSpec((_M, _d, _K), lambda i: (0, 0, 0)),
            pl.BlockSpec((_M, _K), lambda i: (0, 0)),
            pl.BlockSpec((1, 1), lambda i: (0, 0)),
        ],
        out_specs=[
            pl.BlockSpec((_BLK, _M), lambda i: (i, 0)),
            pl.BlockSpec((_BLK, _M), lambda i: (i, 0)),
            pl.BlockSpec((_M, _K), lambda i: (0, 0)),
            pl.BlockSpec((1, 1), lambda i: (0, 0)),
        ] + stats_specs,
        out_shape=[
            jax.ShapeDtypeStruct((_BSEG, _M), jnp.int32),
            jax.ShapeDtypeStruct((_BSEG, _M), jnp.int32),
            jax.ShapeDtypeStruct((_M, _K), jnp.float32),
            jax.ShapeDtypeStruct((1, 1), jnp.float32),
        ] + stats_shapes,
        scratch_shapes=[
            pltpu.VMEM((_M, _K), jnp.float32),
            pltpu.VMEM((_M, _K), jnp.float32),
            pltpu.SMEM((1, 1), jnp.float32),
        ],
    )(z, cbd, cnt_in, csum_in)


@functools.cache
def _make_sc_gather(seg):
    mesh = plsc.VectorSubcoreMesh(core_axis_name="c", subcore_axis_name="s")
    nchunks = _ROWS_W // _CH
    nbuf = 4

    @functools.partial(
        pl.kernel, mesh=mesh,
        out_type=(),
        compiler_params=pltpu.CompilerParams(use_tc_tiling_on_sc=False),
        scratch_types=[
            pltpu.VMEM((_ROWS_W,), jnp.int32),
            [pltpu.VMEM((_CH, _d), jnp.float32) for _ in range(nbuf)],
            [pltpu.SemaphoreType.DMA for _ in range(nbuf)],
            [pltpu.SemaphoreType.DMA for _ in range(nbuf)],
        ],
    )
    def _sc_gather(table_hbm, idx_hbm, out_hbm, idx_v, rows, gsem, osem):
        wid = lax.axis_index("s") * 2 + lax.axis_index("c")
        ibase = wid * _ROWS_W
        obase = seg * _NBSEG + wid * _ROWS_W
        pltpu.sync_copy(idx_hbm.at[pl.ds(ibase, _ROWS_W)], idx_v)

        def gather(c, j):
            return pltpu.async_copy(
                table_hbm.at[idx_v.at[pl.ds(c * _CH, _CH)]], rows[j],
                gsem[j])

        def put(c, j):
            off = pl.multiple_of(obase + c * _CH, _CH)
            return pltpu.async_copy(rows[j], out_hbm.at[pl.ds(off, _CH)],
                                    osem[j])

        g = [gather(j, j) for j in range(nbuf)]
        o = [None] * nbuf
        for c in range(nchunks):
            j = c % nbuf
            g[j].wait()
            o[j] = put(c, j)
            n = c + nbuf
            if n < nchunks:
                o[j].wait()
                g[j] = gather(n, j)
        for j in range(nbuf):
            if o[j] is not None:
                o[j].wait()

    return _sc_gather


def kernel(z, codebook):
    cbd = jnp.transpose(codebook, (0, 2, 1)) * 2.0
    table = codebook.reshape(_M * _K, _d)

    zq_ref = jax.new_ref(jnp.zeros((_NB, _d), jnp.float32))
    cnt = jnp.zeros((_M, _K), jnp.float32)
    cs = jnp.zeros((1, 1), jnp.float32)
    idx_parts = []
    for s in range(_NSEG):
        last = s == _NSEG - 1
        outs = _tc_call(z, cbd, cnt, cs, seg=s, last=last)
        idx_s, fidx_s, cnt, cs = outs[:4]
        idx_parts.append(idx_s)
        if last:
            commit, perp, dead = outs[4:]
        _make_sc_gather(s)(table, fidx_s.reshape(_NBSEG), zq_ref)

    z_st = zq_ref[...].reshape(_B, _D)
    idx = jnp.concatenate(idx_parts, axis=0)
    return (z_st, idx, commit.reshape(()), perp.reshape(_M),
            dead.reshape(_M))
